# Initial kernel scaffold; baseline (speedup 1.0000x reference)
import jax
import jax.numpy as jnp
from jax.experimental import pallas as pl


def kernel(edge_types, edge_embeddings):
    raise NotImplementedError("write your pallas kernel here")


# speedup vs baseline: 7.8910x; 7.8910x over previous
import functools

import jax
import jax.numpy as jnp
from jax import lax
from jax.experimental import pallas as pl
from jax.experimental.pallas import tpu as pltpu
from jax.experimental.pallas import tpu_sc as plsc

_NUM_EDGE_TYPES = 1000
_EDGE_DIM = 16
_N_EDGES = 3200000

_NC = 2
_NS = 16
_NW = _NC * _NS
_B_PER_W = _N_EDGES // _NW
_CHUNK = 4000
_N_CHUNKS = _B_PER_W // _CHUNK

_mesh = plsc.VectorSubcoreMesh(core_axis_name="c", subcore_axis_name="s")


@functools.partial(
    pl.kernel,
    mesh=_mesh,
    out_type=jax.ShapeDtypeStruct((_N_EDGES, _EDGE_DIM), jnp.float32),
    scratch_types=[
        pltpu.VMEM((_CHUNK,), jnp.int32),
        pltpu.VMEM((_CHUNK, _EDGE_DIM), jnp.float32),
        pltpu.SemaphoreType.DMA,
    ],
    compiler_params=pltpu.CompilerParams(use_tc_tiling_on_sc=False),
)
def _gather_kernel(idx_hbm, table_hbm, out_hbm, idx_v, rows_v, sem):
    wid = lax.axis_index("s") * _NC + lax.axis_index("c")
    base = wid * _B_PER_W

    def body(i, carry):
        off = base + i * _CHUNK
        pltpu.sync_copy(idx_hbm.at[pl.ds(off, _CHUNK)], idx_v)
        pltpu.async_copy(table_hbm.at[idx_v], rows_v, sem).wait()
        pltpu.sync_copy(rows_v, out_hbm.at[pl.ds(off, _CHUNK)])
        return carry

    lax.fori_loop(0, _N_CHUNKS, body, 0)


def kernel(edge_types, edge_embeddings):
    return _gather_kernel(edge_types.astype(jnp.int32), edge_embeddings)


# speedup vs baseline: 7.9356x; 1.0057x over previous
import functools

import jax
import jax.numpy as jnp
from jax import lax
from jax.experimental import pallas as pl
from jax.experimental.pallas import tpu as pltpu
from jax.experimental.pallas import tpu_sc as plsc

_NUM_EDGE_TYPES = 1000
_EDGE_DIM = 16
_N_EDGES = 3200000

_NC = 2
_NS = 16
_NW = _NC * _NS
_B_PER_W = _N_EDGES // _NW
_CHUNK = 2000
_N_CHUNKS = _B_PER_W // _CHUNK

_mesh = plsc.VectorSubcoreMesh(core_axis_name="c", subcore_axis_name="s")


@functools.partial(
    pl.kernel,
    mesh=_mesh,
    out_type=jax.ShapeDtypeStruct((_N_EDGES, _EDGE_DIM), jnp.float32),
    scratch_types=[
        pltpu.VMEM((_CHUNK,), jnp.int32),
        pltpu.VMEM((_CHUNK,), jnp.int32),
        pltpu.VMEM((_CHUNK, _EDGE_DIM), jnp.float32),
        pltpu.VMEM((_CHUNK, _EDGE_DIM), jnp.float32),
        pltpu.SemaphoreType.DMA,
        pltpu.SemaphoreType.DMA,
        pltpu.SemaphoreType.DMA,
        pltpu.SemaphoreType.DMA,
        pltpu.SemaphoreType.DMA,
        pltpu.SemaphoreType.DMA,
    ],
    compiler_params=pltpu.CompilerParams(use_tc_tiling_on_sc=False),
)
def _gather_kernel(idx_hbm, table_hbm, out_hbm,
                   idx0, idx1, rows0, rows1, si0, si1, sg0, sg1, ss0, ss1):
    wid = lax.axis_index("s") * _NC + lax.axis_index("c")
    base = wid * _B_PER_W
    last = base + _B_PER_W - _CHUNK
    bufs = ((idx0, rows0, si0, sg0, ss0), (idx1, rows1, si1, sg1, ss1))

    def start_idx(i, idx_b, si_b):
        off = jnp.minimum(base + i * _CHUNK, last)
        pltpu.async_copy(idx_hbm.at[pl.ds(off, _CHUNK)], idx_b, si_b)

    def wait_idx(idx_b, si_b):
        pltpu.make_async_copy(idx_hbm.at[pl.ds(base, _CHUNK)], idx_b, si_b).wait()

    def wait_store(rows_b, ss_b):
        pltpu.make_async_copy(rows_b, out_hbm.at[pl.ds(base, _CHUNK)], ss_b).wait()

    start_idx(0, idx0, si0)
    start_idx(1, idx1, si1)
    for b in range(2):
        idx_b, rows_b, si_b, sg_b, ss_b = bufs[b]
        wait_idx(idx_b, si_b)
        pltpu.async_copy(table_hbm.at[idx_b], rows_b, sg_b).wait()
        pltpu.async_copy(rows_b, out_hbm.at[pl.ds(base + b * _CHUNK, _CHUNK)], ss_b)
        start_idx(b + 2, idx_b, si_b)

    def body(k, carry):
        for b in range(2):
            i = 2 * k + b
            idx_b, rows_b, si_b, sg_b, ss_b = bufs[b]
            wait_idx(idx_b, si_b)
            wait_store(rows_b, ss_b)
            pltpu.async_copy(table_hbm.at[idx_b], rows_b, sg_b).wait()
            pltpu.async_copy(rows_b, out_hbm.at[pl.ds(base + i * _CHUNK, _CHUNK)], ss_b)
            start_idx(i + 2, idx_b, si_b)
        return carry

    lax.fori_loop(1, _N_CHUNKS // 2, body, 0)

    for b in range(2):
        idx_b, rows_b, si_b, sg_b, ss_b = bufs[b]
        wait_store(rows_b, ss_b)
        wait_idx(idx_b, si_b)


def kernel(edge_types, edge_embeddings):
    return _gather_kernel(edge_types.astype(jnp.int32), edge_embeddings)
